# baseline (speedup 1.0000x reference)
import functools

import jax
import jax.numpy as jnp
from jax import lax
from jax.experimental import pallas as pl
from jax.experimental.pallas import tpu as pltpu
from jax.experimental.pallas import tpu_sc as plsc

N = 10000
E = 320000
D = 128

NC = 2
NS = 16
C = 128

NCHUNK = 2 * (-(-E // (NS * C * 2)))
EPT = NCHUNK * C
EPAD = EPT * NS
NPAD = -(-(N + 1) // (NS * 8)) * NS * 8
RPT = NPAD // NS
K0 = 92


def _sc_aggregate(x, src, dst, zeros_init, ones_rows):
  mesh = plsc.VectorSubcoreMesh(
      core_axis_name="c", subcore_axis_name="s", num_cores=NC, num_subcores=NS)

  @functools.partial(
      pl.kernel,
      out_type=(
          jax.ShapeDtypeStruct((NPAD, D), jnp.float32),
          jax.ShapeDtypeStruct((NPAD, D), jnp.float32),
          jax.ShapeDtypeStruct((NPAD, D), jnp.float32),
      ),
      mesh=mesh,
      scratch_types=(
          pltpu.VMEM((C,), jnp.int32),
          (pltpu.VMEM((C,), jnp.int32),) * 2,
          (pltpu.VMEM((C, D), jnp.float32),) * 2,
          pltpu.VMEM((C, D), jnp.float32),
          pltpu.VMEM_SHARED((NPAD, D), jnp.float32),
          pltpu.SemaphoreType.DMA,
          (pltpu.SemaphoreType.DMA,) * 2,
      ),
  )
  def k(x_hbm, src_hbm, dst_hbm, zeros_hbm, ones_hbm,
        agg0_out, agg1_out, deg_out,
        src_v, dst_v, rows_v, ones_v, acc_sh, sem_g, sem_s):
    c = lax.axis_index("c")
    s = lax.axis_index("s")
    tile_base = s * EPT
    row_base = s * RPT

    def zero_acc():
      pltpu.sync_copy(zeros_hbm.at[pl.ds(row_base, RPT)],
                      acc_sh.at[pl.ds(row_base, RPT)])

    def writeback(out_ref):
      pltpu.sync_copy(acc_sh.at[pl.ds(row_base, RPT)],
                      out_ref.at[pl.ds(row_base, RPT)])

    def scatter_desc(b, src_buf):
      return pltpu.make_async_copy(src_buf, acc_sh.at[dst_v[b]], sem_s[b])

    def message_chunks(lo, hi):
      def pair(i, carry):
        for b in (0, 1):
          j = 2 * i + b
          base = tile_base + j * C

          @pl.when(i > lo // 2)
          def _drain():
            scatter_desc(b, rows_v[b]).wait()

          pltpu.sync_copy(src_hbm.at[pl.ds(base, C)], src_v)
          pltpu.sync_copy(dst_hbm.at[pl.ds(base, C)], dst_v[b])
          pltpu.async_copy(x_hbm.at[src_v], rows_v[b], sem_g).wait()
          pltpu.async_copy(rows_v[b], acc_sh.at[dst_v[b]], sem_s[b],
                           add=True)
        return carry

      lax.fori_loop(lo // 2, hi // 2, pair, 0)
      for b in (0, 1):
        scatter_desc(b, rows_v[b]).wait()

    zero_acc()
    pltpu.sync_copy(ones_hbm, ones_v)
    plsc.subcore_barrier()

    @pl.when(c == 0)
    def _sc0():
      message_chunks(0, K0)
      plsc.subcore_barrier()
      writeback(agg0_out)

    @pl.when(c == 1)
    def _sc1():
      def pair(i, carry):
        for b in (0, 1):
          j = 2 * i + b
          base = tile_base + j * C

          @pl.when(i > 0)
          def _drain():
            scatter_desc(b, ones_v).wait()

          pltpu.sync_copy(dst_hbm.at[pl.ds(base, C)], dst_v[b])
          pltpu.async_copy(ones_v, acc_sh.at[dst_v[b]], sem_s[b], add=True)
        return carry

      lax.fori_loop(0, NCHUNK // 2, pair, 0)
      for b in (0, 1):
        scatter_desc(b, ones_v).wait()
      plsc.subcore_barrier()
      writeback(deg_out)
      plsc.subcore_barrier()
      zero_acc()
      plsc.subcore_barrier()
      message_chunks(K0, NCHUNK)
      plsc.subcore_barrier()
      writeback(agg1_out)

  return k(x, src, dst, zeros_init, ones_rows)


def _tc_body(agg0_ref, agg1_ref, deg_ref, x_ref, w_ref, b_ref, o_ref):
  d = deg_ref[...]
  d = jnp.where(d == 0.0, 1.0, d)
  a = (agg0_ref[...] + agg1_ref[...]) / d
  dn = (((1,), (1,)), ((), ()))
  o_ref[...] = (
      lax.dot_general(a, w_ref[...], dn, preferred_element_type=jnp.float32)
      + lax.dot_general(x_ref[...], b_ref[...], dn,
                        preferred_element_type=jnp.float32))


def _tc_update(agg0, agg1, deg, x, W, B):
  blk = 1000
  grid = N // blk
  return pl.pallas_call(
      _tc_body,
      grid=(grid,),
      in_specs=[
          pl.BlockSpec((blk, D), lambda i: (i, 0)),
          pl.BlockSpec((blk, D), lambda i: (i, 0)),
          pl.BlockSpec((blk, D), lambda i: (i, 0)),
          pl.BlockSpec((blk, D), lambda i: (i, 0)),
          pl.BlockSpec((D, D), lambda i: (0, 0)),
          pl.BlockSpec((D, D), lambda i: (0, 0)),
      ],
      out_specs=pl.BlockSpec((blk, D), lambda i: (i, 0)),
      out_shape=jax.ShapeDtypeStruct((N, D), jnp.float32),
  )(agg0, agg1, deg, x, W, B)


def kernel(x, edge_index, W, B):
  src = edge_index[0].astype(jnp.int32)
  dst = edge_index[1].astype(jnp.int32)
  pad = EPAD - E
  src_p = jnp.concatenate([src, jnp.zeros((pad,), jnp.int32)])
  dst_p = jnp.concatenate([dst, jnp.full((pad,), N, jnp.int32)])
  zeros_init = jnp.zeros((NPAD, D), jnp.float32)
  ones_rows = jnp.ones((C, D), jnp.float32)
  agg0, agg1, deg = _sc_aggregate(x, src_p, dst_p, zeros_init, ones_rows)
  return _tc_update(agg0, agg1, deg, x, W, B)


# speedup vs baseline: 1.1629x; 1.1629x over previous
import functools

import jax
import jax.numpy as jnp
from jax import lax
from jax.experimental import pallas as pl
from jax.experimental.pallas import tpu as pltpu
from jax.experimental.pallas import tpu_sc as plsc

N = 10000
E = 320000
D = 128

NC = 2
NS = 16
C = 128

NCHUNK = 2 * (-(-E // (NS * C * 2)))
EPT = NCHUNK * C
EPAD = EPT * NS
NPAD = -(-(N + 1) // (NS * 8)) * NS * 8
RPT = NPAD // NS
K0 = 126


def _sc_aggregate(x, src, dst, zeros_init, ones_rows):
  mesh = plsc.VectorSubcoreMesh(
      core_axis_name="c", subcore_axis_name="s", num_cores=NC, num_subcores=NS)

  @functools.partial(
      pl.kernel,
      out_type=(
          jax.ShapeDtypeStruct((NPAD, D), jnp.float32),
          jax.ShapeDtypeStruct((NPAD, D), jnp.float32),
          jax.ShapeDtypeStruct((NPAD, D), jnp.float32),
      ),
      mesh=mesh,
      scratch_types=(
          pltpu.VMEM((C,), jnp.int32),
          (pltpu.VMEM((C,), jnp.int32),) * 2,
          (pltpu.VMEM((C, D), jnp.float32),) * 2,
          pltpu.VMEM((C, D), jnp.float32),
          pltpu.VMEM_SHARED((NPAD, D), jnp.float32),
          pltpu.SemaphoreType.DMA,
          (pltpu.SemaphoreType.DMA,) * 2,
      ),
  )
  def k(x_hbm, src_hbm, dst_hbm, zeros_hbm, ones_hbm,
        agg0_out, agg1_out, deg_out,
        src_v, dst_v, rows_v, ones_v, acc_sh, sem_g, sem_s):
    c = lax.axis_index("c")
    s = lax.axis_index("s")
    tile_base = s * EPT
    row_base = s * RPT

    def zero_acc():
      pltpu.sync_copy(zeros_hbm.at[pl.ds(row_base, RPT)],
                      acc_sh.at[pl.ds(row_base, RPT)])

    def writeback(out_ref):
      pltpu.sync_copy(acc_sh.at[pl.ds(row_base, RPT)],
                      out_ref.at[pl.ds(row_base, RPT)])

    def scatter_desc(b, src_buf):
      return pltpu.make_async_copy(src_buf, acc_sh.at[dst_v[b]], sem_s[b])

    def message_chunks(lo, hi):
      def pair(i, carry):
        for b in (0, 1):
          j = 2 * i + b
          base = tile_base + j * C

          @pl.when(i > lo // 2)
          def _drain():
            scatter_desc(b, rows_v[b]).wait()

          pltpu.sync_copy(src_hbm.at[pl.ds(base, C)], src_v)
          pltpu.sync_copy(dst_hbm.at[pl.ds(base, C)], dst_v[b])
          pltpu.async_copy(x_hbm.at[src_v], rows_v[b], sem_g).wait()
          pltpu.async_copy(rows_v[b], acc_sh.at[dst_v[b]], sem_s[b],
                           add=True)
        return carry

      lax.fori_loop(lo // 2, hi // 2, pair, 0)
      for b in (0, 1):
        scatter_desc(b, rows_v[b]).wait()

    zero_acc()
    pltpu.sync_copy(ones_hbm, ones_v)
    plsc.subcore_barrier()

    @pl.when(c == 0)
    def _sc0():
      message_chunks(0, K0)
      plsc.subcore_barrier()
      writeback(agg0_out)

    @pl.when(c == 1)
    def _sc1():
      def pair(i, carry):
        for b in (0, 1):
          j = 2 * i + b
          base = tile_base + j * C

          @pl.when(i > 0)
          def _drain():
            scatter_desc(b, ones_v).wait()

          pltpu.sync_copy(dst_hbm.at[pl.ds(base, C)], dst_v[b])
          pltpu.async_copy(ones_v, acc_sh.at[dst_v[b]], sem_s[b], add=True)
        return carry

      lax.fori_loop(0, NCHUNK // 2, pair, 0)
      for b in (0, 1):
        scatter_desc(b, ones_v).wait()
      plsc.subcore_barrier()
      writeback(deg_out)
      plsc.subcore_barrier()
      zero_acc()
      plsc.subcore_barrier()
      message_chunks(K0, NCHUNK)
      plsc.subcore_barrier()
      writeback(agg1_out)

  return k(x, src, dst, zeros_init, ones_rows)


def _tc_body(agg0_ref, agg1_ref, deg_ref, x_ref, w_ref, b_ref, o_ref):
  d = deg_ref[...]
  d = jnp.where(d == 0.0, 1.0, d)
  a = (agg0_ref[...] + agg1_ref[...]) / d
  dn = (((1,), (1,)), ((), ()))
  o_ref[...] = (
      lax.dot_general(a, w_ref[...], dn, preferred_element_type=jnp.float32)
      + lax.dot_general(x_ref[...], b_ref[...], dn,
                        preferred_element_type=jnp.float32))


def _tc_update(agg0, agg1, deg, x, W, B):
  blk = 1000
  grid = N // blk
  return pl.pallas_call(
      _tc_body,
      grid=(grid,),
      in_specs=[
          pl.BlockSpec((blk, D), lambda i: (i, 0)),
          pl.BlockSpec((blk, D), lambda i: (i, 0)),
          pl.BlockSpec((blk, D), lambda i: (i, 0)),
          pl.BlockSpec((blk, D), lambda i: (i, 0)),
          pl.BlockSpec((D, D), lambda i: (0, 0)),
          pl.BlockSpec((D, D), lambda i: (0, 0)),
      ],
      out_specs=pl.BlockSpec((blk, D), lambda i: (i, 0)),
      out_shape=jax.ShapeDtypeStruct((N, D), jnp.float32),
  )(agg0, agg1, deg, x, W, B)


def kernel(x, edge_index, W, B):
  src = edge_index[0].astype(jnp.int32)
  dst = edge_index[1].astype(jnp.int32)
  pad = EPAD - E
  src_p = jnp.concatenate([src, jnp.zeros((pad,), jnp.int32)])
  dst_p = jnp.concatenate([dst, jnp.full((pad,), N, jnp.int32)])
  zeros_init = jnp.zeros((NPAD, D), jnp.float32)
  ones_rows = jnp.ones((C, D), jnp.float32)
  agg0, agg1, deg = _sc_aggregate(x, src_p, dst_p, zeros_init, ones_rows)
  return _tc_update(agg0, agg1, deg, x, W, B)


# speedup vs baseline: 1.1640x; 1.0009x over previous
import functools

import jax
import jax.numpy as jnp
from jax import lax
from jax.experimental import pallas as pl
from jax.experimental.pallas import tpu as pltpu
from jax.experimental.pallas import tpu_sc as plsc

N = 10000
E = 320000
D = 128

NC = 2
NS = 16
C = 128

NCHUNK = 2 * (-(-E // (NS * C * 2)))
EPT = NCHUNK * C
EPAD = EPT * NS
NPAD = -(-(N + 1) // (NS * 8)) * NS * 8
RPT = NPAD // NS
K0 = 140


def _sc_aggregate(x, src, dst, zeros_init, ones_rows):
  mesh = plsc.VectorSubcoreMesh(
      core_axis_name="c", subcore_axis_name="s", num_cores=NC, num_subcores=NS)

  @functools.partial(
      pl.kernel,
      out_type=(
          jax.ShapeDtypeStruct((NPAD, D), jnp.float32),
          jax.ShapeDtypeStruct((NPAD, D), jnp.float32),
          jax.ShapeDtypeStruct((NPAD, D), jnp.float32),
      ),
      mesh=mesh,
      scratch_types=(
          pltpu.VMEM((C,), jnp.int32),
          (pltpu.VMEM((C,), jnp.int32),) * 2,
          (pltpu.VMEM((C, D), jnp.float32),) * 2,
          pltpu.VMEM((C, D), jnp.float32),
          pltpu.VMEM_SHARED((NPAD, D), jnp.float32),
          pltpu.SemaphoreType.DMA,
          (pltpu.SemaphoreType.DMA,) * 2,
      ),
  )
  def k(x_hbm, src_hbm, dst_hbm, zeros_hbm, ones_hbm,
        agg0_out, agg1_out, deg_out,
        src_v, dst_v, rows_v, ones_v, acc_sh, sem_g, sem_s):
    c = lax.axis_index("c")
    s = lax.axis_index("s")
    tile_base = s * EPT
    row_base = s * RPT

    def zero_acc():
      pltpu.sync_copy(zeros_hbm.at[pl.ds(row_base, RPT)],
                      acc_sh.at[pl.ds(row_base, RPT)])

    def writeback(out_ref):
      pltpu.sync_copy(acc_sh.at[pl.ds(row_base, RPT)],
                      out_ref.at[pl.ds(row_base, RPT)])

    def scatter_desc(b, src_buf):
      return pltpu.make_async_copy(src_buf, acc_sh.at[dst_v[b]], sem_s[b])

    def message_chunks(lo, hi):
      def pair(i, carry):
        for b in (0, 1):
          j = 2 * i + b
          base = tile_base + j * C

          @pl.when(i > lo // 2)
          def _drain():
            scatter_desc(b, rows_v[b]).wait()

          pltpu.sync_copy(src_hbm.at[pl.ds(base, C)], src_v)
          pltpu.sync_copy(dst_hbm.at[pl.ds(base, C)], dst_v[b])
          pltpu.async_copy(x_hbm.at[src_v], rows_v[b], sem_g).wait()
          pltpu.async_copy(rows_v[b], acc_sh.at[dst_v[b]], sem_s[b],
                           add=True)
        return carry

      lax.fori_loop(lo // 2, hi // 2, pair, 0)
      for b in (0, 1):
        scatter_desc(b, rows_v[b]).wait()

    zero_acc()
    pltpu.sync_copy(ones_hbm, ones_v)
    plsc.subcore_barrier()

    @pl.when(c == 0)
    def _sc0():
      message_chunks(0, K0)
      plsc.subcore_barrier()
      writeback(agg0_out)

    @pl.when(c == 1)
    def _sc1():
      def pair(i, carry):
        for b in (0, 1):
          j = 2 * i + b
          base = tile_base + j * C

          @pl.when(i > 0)
          def _drain():
            scatter_desc(b, ones_v).wait()

          pltpu.sync_copy(dst_hbm.at[pl.ds(base, C)], dst_v[b])
          pltpu.async_copy(ones_v, acc_sh.at[dst_v[b]], sem_s[b], add=True)
        return carry

      lax.fori_loop(0, NCHUNK // 2, pair, 0)
      for b in (0, 1):
        scatter_desc(b, ones_v).wait()
      plsc.subcore_barrier()
      writeback(deg_out)
      plsc.subcore_barrier()
      zero_acc()
      plsc.subcore_barrier()
      message_chunks(K0, NCHUNK)
      plsc.subcore_barrier()
      writeback(agg1_out)

  return k(x, src, dst, zeros_init, ones_rows)


def _tc_body(agg0_ref, agg1_ref, deg_ref, x_ref, w_ref, b_ref, o_ref):
  d = deg_ref[...]
  d = jnp.where(d == 0.0, 1.0, d)
  a = (agg0_ref[...] + agg1_ref[...]) / d
  dn = (((1,), (1,)), ((), ()))
  o_ref[...] = (
      lax.dot_general(a, w_ref[...], dn, preferred_element_type=jnp.float32)
      + lax.dot_general(x_ref[...], b_ref[...], dn,
                        preferred_element_type=jnp.float32))


def _tc_update(agg0, agg1, deg, x, W, B):
  blk = 1000
  grid = N // blk
  return pl.pallas_call(
      _tc_body,
      grid=(grid,),
      in_specs=[
          pl.BlockSpec((blk, D), lambda i: (i, 0)),
          pl.BlockSpec((blk, D), lambda i: (i, 0)),
          pl.BlockSpec((blk, D), lambda i: (i, 0)),
          pl.BlockSpec((blk, D), lambda i: (i, 0)),
          pl.BlockSpec((D, D), lambda i: (0, 0)),
          pl.BlockSpec((D, D), lambda i: (0, 0)),
      ],
      out_specs=pl.BlockSpec((blk, D), lambda i: (i, 0)),
      out_shape=jax.ShapeDtypeStruct((N, D), jnp.float32),
  )(agg0, agg1, deg, x, W, B)


def kernel(x, edge_index, W, B):
  src = edge_index[0].astype(jnp.int32)
  dst = edge_index[1].astype(jnp.int32)
  pad = EPAD - E
  src_p = jnp.concatenate([src, jnp.zeros((pad,), jnp.int32)])
  dst_p = jnp.concatenate([dst, jnp.full((pad,), N, jnp.int32)])
  zeros_init = jnp.zeros((NPAD, D), jnp.float32)
  ones_rows = jnp.ones((C, D), jnp.float32)
  agg0, agg1, deg = _sc_aggregate(x, src_p, dst_p, zeros_init, ones_rows)
  return _tc_update(agg0, agg1, deg, x, W, B)
